# initial kernel scaffold (unmeasured)
import jax
import jax.numpy as jnp
from jax import lax
from jax.experimental import pallas as pl
from jax.experimental.pallas import tpu as pltpu

N_DEV = 4
M_PER = 1024
K_BLK = 1024
N_TOTAL = 8192
N_CHUNK = 1024
NB = N_TOTAL // N_CHUNK


def kernel(x, w_mat):
    x = x.astype(jnp.bfloat16)

    def body(x_ref, w_ref, out_ref, xg_ref, wbuf_ref, amax_ref,
             send_sems, recv_sems, a_send_sems, a_recv_sems, w_sems):
        my = lax.axis_index("i")

        barrier = pltpu.get_barrier_semaphore()
        for dj in range(1, N_DEV):
            pl.semaphore_signal(
                barrier, inc=1,
                device_id=((my + dj) % N_DEV,),
                device_id_type=pl.DeviceIdType.MESH,
            )
        pl.semaphore_wait(barrier, N_DEV - 1)

        x_sends = []
        for dj in range(1, N_DEV):
            j = (my + dj) % N_DEV
            rdma = pltpu.make_async_remote_copy(
                src_ref=x_ref.at[pl.ds(j * M_PER, M_PER), :],
                dst_ref=xg_ref.at[dj - 1],
                send_sem=send_sems.at[dj - 1],
                recv_sem=recv_sems.at[dj - 1],
                device_id=(j,),
                device_id_type=pl.DeviceIdType.MESH,
            )
            rdma.start()
            x_sends.append(rdma)

        def recv_done(slot):
            return pltpu.make_async_remote_copy(
                src_ref=xg_ref.at[slot],
                dst_ref=xg_ref.at[slot],
                send_sem=send_sems.at[slot],
                recv_sem=recv_sems.at[slot],
                device_id=(my,),
                device_id_type=pl.DeviceIdType.MESH,
            )

        k_seq = [
            (None, my * K_BLK),
            (0, ((my - 1) % N_DEV) * K_BLK),
            (2, ((my + 1) % N_DEV) * K_BLK),
            (1, ((my + 2) % N_DEV) * K_BLK),
        ]
        flat = [(ki, nb) for ki in range(N_DEV) for nb in range(NB)]

        w_descs = {}

        def start_w(step):
            ki, nb = flat[step]
            _, krow = k_seq[ki]
            d = pltpu.make_async_copy(
                w_ref.at[pl.ds(krow, K_BLK), pl.ds(nb * N_CHUNK, N_CHUNK)],
                wbuf_ref.at[step % 2],
                w_sems.at[step % 2],
            )
            d.start()
            w_descs[step] = d

        start_w(0)
        amax = jnp.float32(0.0)
        for step, (ki, nb) in enumerate(flat):
            slot, _ = k_seq[ki]
            if slot is not None and nb == 0:
                recv_done(slot).wait_recv()
            if step + 1 < len(flat):
                start_w(step + 1)
            w_descs[step].wait()
            wblk = wbuf_ref[step % 2].astype(jnp.bfloat16)
            a = x_ref[pl.ds(my * M_PER, M_PER), :] if slot is None else xg_ref[slot]
            contrib = jnp.dot(a, wblk, preferred_element_type=jnp.float32)
            nsl = pl.ds(nb * N_CHUNK, N_CHUNK)
            if ki == 0:
                out_ref[:, nsl] = contrib
            else:
                acc = out_ref[:, nsl] + contrib
                out_ref[:, nsl] = acc
                if ki == N_DEV - 1:
                    amax = jnp.maximum(amax, jnp.max(jnp.abs(acc)))

        for rdma in x_sends:
            rdma.wait_send()

        amax_ref[N_DEV - 1] = jnp.full((8, 128), amax, jnp.float32)
        a_sends = []
        for dj in range(1, N_DEV):
            j = (my + dj) % N_DEV
            rdma = pltpu.make_async_remote_copy(
                src_ref=amax_ref.at[N_DEV - 1],
                dst_ref=amax_ref.at[dj - 1],
                send_sem=a_send_sems.at[dj - 1],
                recv_sem=a_recv_sems.at[dj - 1],
                device_id=(j,),
                device_id_type=pl.DeviceIdType.MESH,
            )
            rdma.start()
            a_sends.append(rdma)
        for s in range(N_DEV - 1):
            pltpu.make_async_remote_copy(
                src_ref=amax_ref.at[s],
                dst_ref=amax_ref.at[s],
                send_sem=a_send_sems.at[s],
                recv_sem=a_recv_sems.at[s],
                device_id=(my,),
                device_id_type=pl.DeviceIdType.MESH,
            ).wait_recv()
        g = jnp.max(amax_ref[...])

        scale = g / 448.0
        inv = 1.0 / scale
        for nb in range(NB):
            nsl = pl.ds(nb * N_CHUNK, N_CHUNK)
            v = out_ref[:, nsl]
            q = jnp.clip(v * inv, -448.0, 448.0).astype(jnp.float8_e4m3fn)
            out_ref[:, nsl] = q.astype(jnp.float32) * scale

        for rdma in a_sends:
            rdma.wait_send()

    return pl.pallas_call(
        body,
        out_shape=jax.ShapeDtypeStruct((M_PER, N_TOTAL), jnp.float32),
        in_specs=[
            pl.BlockSpec(memory_space=pltpu.VMEM),
            pl.BlockSpec(memory_space=pltpu.ANY),
        ],
        out_specs=pl.BlockSpec(memory_space=pltpu.VMEM),
        scratch_shapes=[
            pltpu.VMEM((N_DEV - 1, M_PER, K_BLK), jnp.bfloat16),
            pltpu.VMEM((2, K_BLK, N_CHUNK), jnp.float32),
            pltpu.VMEM((N_DEV, 8, 128), jnp.float32),
            pltpu.SemaphoreType.DMA((N_DEV - 1,)),
            pltpu.SemaphoreType.DMA((N_DEV - 1,)),
            pltpu.SemaphoreType.DMA((N_DEV - 1,)),
            pltpu.SemaphoreType.DMA((N_DEV - 1,)),
            pltpu.SemaphoreType.DMA((2,)),
        ],
        compiler_params=pltpu.CompilerParams(collective_id=0),
    )(x, w_mat)


# baseline (device time: 153329 ns/iter reference)
import jax
import jax.numpy as jnp
from jax import lax
from jax.experimental import pallas as pl
from jax.experimental.pallas import tpu as pltpu

N_DEV = 4
M_PER = 1024
K_BLK = 1024
N_TOTAL = 8192
N_CHUNK = 1024
NB = N_TOTAL // N_CHUNK


def kernel(x, w_mat):
    x = x.astype(jnp.bfloat16)

    def body(x_ref, w_ref, out_ref, xg_ref, wbuf_ref, amax_ref,
             send_sems, recv_sems, a_send_sems, a_recv_sems, w_sems):
        my = lax.axis_index("i")

        barrier = pltpu.get_barrier_semaphore()
        for dj in range(1, N_DEV):
            pl.semaphore_signal(
                barrier, inc=1,
                device_id=((my + dj) % N_DEV,),
                device_id_type=pl.DeviceIdType.MESH,
            )
        pl.semaphore_wait(barrier, N_DEV - 1)

        x_sends = []
        for dj in range(1, N_DEV):
            j = (my + dj) % N_DEV
            rdma = pltpu.make_async_remote_copy(
                src_ref=x_ref.at[pl.ds(j * M_PER, M_PER), :],
                dst_ref=xg_ref.at[dj - 1],
                send_sem=send_sems.at[dj - 1],
                recv_sem=recv_sems.at[dj - 1],
                device_id=(j,),
                device_id_type=pl.DeviceIdType.MESH,
            )
            rdma.start()
            x_sends.append(rdma)

        def recv_done(slot):
            return pltpu.make_async_remote_copy(
                src_ref=xg_ref.at[slot],
                dst_ref=xg_ref.at[slot],
                send_sem=send_sems.at[slot],
                recv_sem=recv_sems.at[slot],
                device_id=(my,),
                device_id_type=pl.DeviceIdType.MESH,
            )

        k_seq = [
            (None, my * K_BLK),
            (0, ((my - 1) % N_DEV) * K_BLK),
            (2, ((my + 1) % N_DEV) * K_BLK),
            (1, ((my + 2) % N_DEV) * K_BLK),
        ]
        flat = [(ki, nb) for ki in range(N_DEV) for nb in range(NB)]

        w_descs = {}

        def start_w(step):
            ki, nb = flat[step]
            _, krow = k_seq[ki]
            d = pltpu.make_async_copy(
                w_ref.at[pl.ds(krow, K_BLK), pl.ds(nb * N_CHUNK, N_CHUNK)],
                wbuf_ref.at[step % 2],
                w_sems.at[step % 2],
            )
            d.start()
            w_descs[step] = d

        start_w(0)
        amax = jnp.float32(0.0)
        for step, (ki, nb) in enumerate(flat):
            slot, _ = k_seq[ki]
            if slot is not None and nb == 0:
                recv_done(slot).wait_recv()
            if step + 1 < len(flat):
                start_w(step + 1)
            w_descs[step].wait()
            wblk = wbuf_ref[step % 2].astype(jnp.bfloat16)
            a = x_ref[pl.ds(my * M_PER, M_PER), :] if slot is None else xg_ref[slot]
            contrib = jnp.dot(a, wblk, preferred_element_type=jnp.float32)
            nsl = pl.ds(nb * N_CHUNK, N_CHUNK)
            if ki == 0:
                out_ref[:, nsl] = contrib
            else:
                acc = out_ref[:, nsl] + contrib
                out_ref[:, nsl] = acc
                if ki == N_DEV - 1:
                    amax = jnp.maximum(amax, jnp.max(jnp.abs(acc)))

        for rdma in x_sends:
            rdma.wait_send()

        amax_ref[N_DEV - 1] = jnp.full((8, 128), amax, jnp.float32)
        a_sends = []
        for dj in range(1, N_DEV):
            j = (my + dj) % N_DEV
            rdma = pltpu.make_async_remote_copy(
                src_ref=amax_ref.at[N_DEV - 1],
                dst_ref=amax_ref.at[dj - 1],
                send_sem=a_send_sems.at[dj - 1],
                recv_sem=a_recv_sems.at[dj - 1],
                device_id=(j,),
                device_id_type=pl.DeviceIdType.MESH,
            )
            rdma.start()
            a_sends.append(rdma)
        for s in range(N_DEV - 1):
            pltpu.make_async_remote_copy(
                src_ref=amax_ref.at[s],
                dst_ref=amax_ref.at[s],
                send_sem=a_send_sems.at[s],
                recv_sem=a_recv_sems.at[s],
                device_id=(my,),
                device_id_type=pl.DeviceIdType.MESH,
            ).wait_recv()
        g = jnp.max(amax_ref[...])

        scale = g / 448.0
        inv = 1.0 / scale
        for nb in range(NB):
            nsl = pl.ds(nb * N_CHUNK, N_CHUNK)
            v = out_ref[:, nsl]
            q = jnp.clip(v * inv, -448.0, 448.0).astype(jnp.float8_e4m3fn)
            out_ref[:, nsl] = q.astype(jnp.float32) * scale

        for rdma in a_sends:
            rdma.wait_send()

    return pl.pallas_call(
        body,
        out_shape=jax.ShapeDtypeStruct((M_PER, N_TOTAL), jnp.float32),
        in_specs=[
            pl.BlockSpec(memory_space=pltpu.VMEM),
            pl.BlockSpec(memory_space=pl.ANY),
        ],
        out_specs=pl.BlockSpec(memory_space=pltpu.VMEM),
        scratch_shapes=[
            pltpu.VMEM((N_DEV - 1, M_PER, K_BLK), jnp.bfloat16),
            pltpu.VMEM((2, K_BLK, N_CHUNK), jnp.float32),
            pltpu.VMEM((N_DEV, 8, 128), jnp.float32),
            pltpu.SemaphoreType.DMA((N_DEV - 1,)),
            pltpu.SemaphoreType.DMA((N_DEV - 1,)),
            pltpu.SemaphoreType.DMA((N_DEV - 1,)),
            pltpu.SemaphoreType.DMA((N_DEV - 1,)),
            pltpu.SemaphoreType.DMA((2,)),
        ],
        compiler_params=pltpu.CompilerParams(
            collective_id=0, vmem_limit_bytes=100 * 1024 * 1024
        ),
    )(x, w_mat)


# device time: 146976 ns/iter; 1.0432x vs baseline; 1.0432x over previous
import jax
import jax.numpy as jnp
from jax import lax
from jax.experimental import pallas as pl
from jax.experimental.pallas import tpu as pltpu

N_DEV = 4
M_PER = 1024
K_BLK = 1024
N_TOTAL = 8192
N_CHUNK = 1024
NB = N_TOTAL // N_CHUNK


def kernel(x, w_mat):
    x = x.astype(jnp.bfloat16)

    def body(x_ref, w_ref, out_ref, xg_ref, wbuf_ref, amax_ref,
             send_sems, recv_sems, a_send_sems, a_recv_sems, w_sems):
        my = lax.axis_index("i")

        k_seq = [
            (None, my * K_BLK),
            (0, ((my - 1) % N_DEV) * K_BLK),
            (2, ((my + 1) % N_DEV) * K_BLK),
            (1, ((my + 2) % N_DEV) * K_BLK),
        ]
        flat = [(ki, nb) for ki in range(N_DEV) for nb in range(NB)]
        NSLOT = 3
        w_descs = {}

        def start_w(step):
            ki, nb = flat[step]
            _, krow = k_seq[ki]
            d = pltpu.make_async_copy(
                w_ref.at[pl.ds(krow, K_BLK), pl.ds(nb * N_CHUNK, N_CHUNK)],
                wbuf_ref.at[step % NSLOT],
                w_sems.at[step % NSLOT],
            )
            d.start()
            w_descs[step] = d

        start_w(0)
        start_w(1)

        barrier = pltpu.get_barrier_semaphore()
        for dj in range(1, N_DEV):
            pl.semaphore_signal(
                barrier, inc=1,
                device_id=((my + dj) % N_DEV,),
                device_id_type=pl.DeviceIdType.MESH,
            )
        pl.semaphore_wait(barrier, N_DEV - 1)

        x_sends = []
        for dj in range(1, N_DEV):
            j = (my + dj) % N_DEV
            rdma = pltpu.make_async_remote_copy(
                src_ref=x_ref.at[pl.ds(j * M_PER, M_PER), :],
                dst_ref=xg_ref.at[dj - 1],
                send_sem=send_sems.at[dj - 1],
                recv_sem=recv_sems.at[dj - 1],
                device_id=(j,),
                device_id_type=pl.DeviceIdType.MESH,
            )
            rdma.start()
            x_sends.append(rdma)

        def recv_done(slot):
            return pltpu.make_async_remote_copy(
                src_ref=xg_ref.at[slot],
                dst_ref=xg_ref.at[slot],
                send_sem=send_sems.at[slot],
                recv_sem=recv_sems.at[slot],
                device_id=(my,),
                device_id_type=pl.DeviceIdType.MESH,
            )

        amax = jnp.float32(0.0)
        for step, (ki, nb) in enumerate(flat):
            slot, _ = k_seq[ki]
            if slot is not None and nb == 0:
                recv_done(slot).wait_recv()
            if nb == 0:
                a = (x_ref[pl.ds(my * M_PER, M_PER), :] if slot is None
                     else xg_ref[slot]).astype(jnp.float32)
            if step + 2 < len(flat):
                start_w(step + 2)
            w_descs[step].wait()
            contrib = jnp.dot(a, wbuf_ref[step % NSLOT],
                              preferred_element_type=jnp.float32)
            nsl = pl.ds(nb * N_CHUNK, N_CHUNK)
            if ki == 0:
                out_ref[:, nsl] = contrib
            else:
                acc = out_ref[:, nsl] + contrib
                out_ref[:, nsl] = acc
                if ki == N_DEV - 1:
                    amax = jnp.maximum(amax, jnp.max(jnp.abs(acc)))

        for rdma in x_sends:
            rdma.wait_send()

        amax_ref[N_DEV - 1] = jnp.full((8, 128), amax, jnp.float32)
        a_sends = []
        for dj in range(1, N_DEV):
            j = (my + dj) % N_DEV
            rdma = pltpu.make_async_remote_copy(
                src_ref=amax_ref.at[N_DEV - 1],
                dst_ref=amax_ref.at[dj - 1],
                send_sem=a_send_sems.at[dj - 1],
                recv_sem=a_recv_sems.at[dj - 1],
                device_id=(j,),
                device_id_type=pl.DeviceIdType.MESH,
            )
            rdma.start()
            a_sends.append(rdma)
        for s in range(N_DEV - 1):
            pltpu.make_async_remote_copy(
                src_ref=amax_ref.at[s],
                dst_ref=amax_ref.at[s],
                send_sem=a_send_sems.at[s],
                recv_sem=a_recv_sems.at[s],
                device_id=(my,),
                device_id_type=pl.DeviceIdType.MESH,
            ).wait_recv()
        g = jnp.max(amax_ref[...])

        scale = g / 448.0
        inv = 1.0 / scale
        for nb in range(NB):
            nsl = pl.ds(nb * N_CHUNK, N_CHUNK)
            v = out_ref[:, nsl]
            q = jnp.clip(v * inv, -448.0, 448.0).astype(jnp.float8_e4m3fn)
            out_ref[:, nsl] = q.astype(jnp.float32) * scale

        for rdma in a_sends:
            rdma.wait_send()

    return pl.pallas_call(
        body,
        out_shape=jax.ShapeDtypeStruct((M_PER, N_TOTAL), jnp.float32),
        in_specs=[
            pl.BlockSpec(memory_space=pltpu.VMEM),
            pl.BlockSpec(memory_space=pl.ANY),
        ],
        out_specs=pl.BlockSpec(memory_space=pltpu.VMEM),
        scratch_shapes=[
            pltpu.VMEM((N_DEV - 1, M_PER, K_BLK), jnp.bfloat16),
            pltpu.VMEM((3, K_BLK, N_CHUNK), jnp.float32),
            pltpu.VMEM((N_DEV, 8, 128), jnp.float32),
            pltpu.SemaphoreType.DMA((N_DEV - 1,)),
            pltpu.SemaphoreType.DMA((N_DEV - 1,)),
            pltpu.SemaphoreType.DMA((N_DEV - 1,)),
            pltpu.SemaphoreType.DMA((N_DEV - 1,)),
            pltpu.SemaphoreType.DMA((3,)),
        ],
        compiler_params=pltpu.CompilerParams(
            collective_id=0, vmem_limit_bytes=100 * 1024 * 1024
        ),
    )(x, w_mat)


# device time: 146189 ns/iter; 1.0488x vs baseline; 1.0054x over previous
import os

import jax
import jax.numpy as jnp
from jax import lax
from jax.experimental import pallas as pl
from jax.experimental.pallas import tpu as pltpu

_NO_COMM = bool(int(os.environ.get("A2A_NO_COMM", "0")))
_NO_EPI = bool(int(os.environ.get("A2A_NO_EPI", "0")))

N_DEV = 4
M_PER = 1024
K_BLK = 1024
N_TOTAL = 8192
N_CHUNK = 1024
NB = N_TOTAL // N_CHUNK


def kernel(x, w_mat):
    def body(x_ref, w_ref, out_ref, xstage_ref, xsend_ref, xg_ref, wbuf_ref,
             amax_ref, send_sems, recv_sems, a_send_sems, a_recv_sems,
             w_sems, x_sems):
        my = lax.axis_index("i")

        k_seq = [
            (None, my * K_BLK),
            (0, ((my - 1) % N_DEV) * K_BLK),
            (2, ((my + 1) % N_DEV) * K_BLK),
            (1, ((my + 2) % N_DEV) * K_BLK),
        ]
        flat = [(ki, nb) for ki in range(N_DEV) for nb in range(NB)]
        NSLOT = 2
        w_descs = {}

        def start_w(step):
            ki, nb = flat[step]
            _, krow = k_seq[ki]
            d = pltpu.make_async_copy(
                w_ref.at[pl.ds(krow, K_BLK), pl.ds(nb * N_CHUNK, N_CHUNK)],
                wbuf_ref.at[step % NSLOT],
                w_sems.at[step % NSLOT],
            )
            d.start()
            w_descs[step] = d

        def start_x(block_j):
            d = pltpu.make_async_copy(
                x_ref.at[pl.ds(block_j * M_PER, M_PER), :],
                xstage_ref,
                x_sems,
            )
            d.start()
            return d

        xd1 = start_x((my + 1) % N_DEV)
        start_w(0)
        start_w(1)

        if not _NO_COMM:
            barrier = pltpu.get_barrier_semaphore()
            for dj in range(1, N_DEV):
                pl.semaphore_signal(
                    barrier, inc=1,
                    device_id=((my + dj) % N_DEV,),
                    device_id_type=pl.DeviceIdType.MESH,
                )
            pl.semaphore_wait(barrier, N_DEV - 1)

        x_sends = []

        def send_block(dj):
            j = (my + dj) % N_DEV
            rdma = pltpu.make_async_remote_copy(
                src_ref=xsend_ref.at[dj - 1],
                dst_ref=xg_ref.at[dj - 1],
                send_sem=send_sems.at[dj - 1],
                recv_sem=recv_sems.at[dj - 1],
                device_id=(j,),
                device_id_type=pl.DeviceIdType.MESH,
            )
            rdma.start()
            x_sends.append(rdma)

        xd1.wait()
        xsend_ref[0] = xstage_ref[...].astype(jnp.bfloat16)
        xd2 = start_x((my + 2) % N_DEV)
        if not _NO_COMM:
            send_block(1)
        xd2.wait()
        xsend_ref[1] = xstage_ref[...].astype(jnp.bfloat16)
        xd3 = start_x((my + 3) % N_DEV)
        if not _NO_COMM:
            send_block(2)
        xd3.wait()
        xsend_ref[2] = xstage_ref[...].astype(jnp.bfloat16)
        if not _NO_COMM:
            send_block(3)
        xdl = start_x(my)
        xdl.wait()

        def recv_done(slot):
            return pltpu.make_async_remote_copy(
                src_ref=xg_ref.at[slot],
                dst_ref=xg_ref.at[slot],
                send_sem=send_sems.at[slot],
                recv_sem=recv_sems.at[slot],
                device_id=(my,),
                device_id_type=pl.DeviceIdType.MESH,
            )

        amax = jnp.float32(0.0)
        for step, (ki, nb) in enumerate(flat):
            slot, _ = k_seq[ki]
            if not _NO_COMM and slot is not None and nb == 0:
                recv_done(slot).wait_recv()
            if nb == 0:
                if slot is None or _NO_COMM:
                    a = xstage_ref[...]
                else:
                    a = xg_ref[slot].astype(jnp.float32)
            if step + 1 < len(flat) and step + 1 not in w_descs:
                start_w(step + 1)
            w_descs[step].wait()
            contrib = jnp.dot(a, wbuf_ref[step % NSLOT],
                              preferred_element_type=jnp.float32)
            nsl = pl.ds(nb * N_CHUNK, N_CHUNK)
            if ki == 0:
                out_ref[:, nsl] = contrib
            else:
                acc = out_ref[:, nsl] + contrib
                out_ref[:, nsl] = acc
                if ki == N_DEV - 1:
                    amax = jnp.maximum(amax, jnp.max(jnp.abs(acc)))

        for rdma in x_sends:
            rdma.wait_send()

        a_sends = []
        if _NO_COMM or _NO_EPI:
            g = amax
        else:
            amax_ref[N_DEV - 1] = jnp.full((8, 128), amax, jnp.float32)
            for dj in range(1, N_DEV):
                j = (my + dj) % N_DEV
                rdma = pltpu.make_async_remote_copy(
                    src_ref=amax_ref.at[N_DEV - 1],
                    dst_ref=amax_ref.at[dj - 1],
                    send_sem=a_send_sems.at[dj - 1],
                    recv_sem=a_recv_sems.at[dj - 1],
                    device_id=(j,),
                    device_id_type=pl.DeviceIdType.MESH,
                )
                rdma.start()
                a_sends.append(rdma)
            for s in range(N_DEV - 1):
                pltpu.make_async_remote_copy(
                    src_ref=amax_ref.at[s],
                    dst_ref=amax_ref.at[s],
                    send_sem=a_send_sems.at[s],
                    recv_sem=a_recv_sems.at[s],
                    device_id=(my,),
                    device_id_type=pl.DeviceIdType.MESH,
                ).wait_recv()
            g = jnp.max(amax_ref[...])

        if not _NO_EPI:
            scale = g / 448.0
            inv = 1.0 / scale
            for nb in range(NB):
                nsl = pl.ds(nb * N_CHUNK, N_CHUNK)
                v = out_ref[:, nsl]
                q = jnp.clip(v * inv, -448.0, 448.0).astype(jnp.float8_e4m3fn)
                out_ref[:, nsl] = q.astype(jnp.float32) * scale

        for rdma in a_sends:
            rdma.wait_send()

    return pl.pallas_call(
        body,
        out_shape=jax.ShapeDtypeStruct((M_PER, N_TOTAL), jnp.float32),
        in_specs=[
            pl.BlockSpec(memory_space=pl.ANY),
            pl.BlockSpec(memory_space=pl.ANY),
        ],
        out_specs=pl.BlockSpec(memory_space=pltpu.VMEM),
        scratch_shapes=[
            pltpu.VMEM((M_PER, K_BLK), jnp.float32),
            pltpu.VMEM((N_DEV - 1, M_PER, K_BLK), jnp.bfloat16),
            pltpu.VMEM((N_DEV - 1, M_PER, K_BLK), jnp.bfloat16),
            pltpu.VMEM((2, K_BLK, N_CHUNK), jnp.float32),
            pltpu.VMEM((N_DEV, 8, 128), jnp.float32),
            pltpu.SemaphoreType.DMA((N_DEV - 1,)),
            pltpu.SemaphoreType.DMA((N_DEV - 1,)),
            pltpu.SemaphoreType.DMA((N_DEV - 1,)),
            pltpu.SemaphoreType.DMA((N_DEV - 1,)),
            pltpu.SemaphoreType.DMA((2,)),
            pltpu.SemaphoreType.DMA,
        ],
        compiler_params=pltpu.CompilerParams(
            vmem_limit_bytes=100 * 1024 * 1024,
            **({} if _NO_COMM else {"collective_id": 0}),
        ),
    )(x, w_mat)
